# Initial kernel scaffold; baseline (speedup 1.0000x reference)
import jax
import jax.numpy as jnp
from jax.experimental import pallas as pl


def kernel(x, W, class_token, pos_encoding):
    raise NotImplementedError("write your pallas kernel here")


# speedup vs baseline: 3.1685x; 3.1685x over previous
import functools
import math

import jax
import jax.numpy as jnp
from jax import lax
from jax.experimental import pallas as pl
from jax.experimental.pallas import tpu as pltpu, tpu_sc as plsc

B = 1024
L = 200
D = 128
SCALE = math.sqrt(float(D))

_NC = 2
_NS = 16
_NW = _NC * _NS
_BPW = B // _NW
_HALF = L // 2


def _sc_body(w_hbm, xi_hbm, pos_hbm, ct_hbm, out_hbm,
             tmpl_v, ct_v, idx_v, obuf_v, gsem):
    wid = lax.axis_index("s") * _NC + lax.axis_index("c")

    pltpu.sync_copy(pos_hbm, tmpl_v)
    pltpu.sync_copy(ct_hbm, ct_v)
    for c in range(D // 16):
        sl = pl.ds(c * 16, 16)
        tmpl_v[0, sl] = tmpl_v[0, sl] + ct_v[sl]
        obuf_v[0, sl] = tmpl_v[0, sl]

    def per_b(i, carry):
        b = wid * _BPW + i
        pltpu.sync_copy(xi_hbm.at[b], idx_v)
        g0 = pltpu.async_copy(w_hbm.at[idx_v.at[0]],
                              obuf_v.at[pl.ds(1, _HALF)], gsem)
        g1 = pltpu.async_copy(w_hbm.at[idx_v.at[1]],
                              obuf_v.at[pl.ds(1 + _HALF, _HALF)], gsem)
        g0.wait()
        g1.wait()

        def row_body(r, c2):
            rr = r + 1
            for c in range(D // 16):
                sl = pl.ds(c * 16, 16)
                obuf_v[rr, sl] = obuf_v[rr, sl] * SCALE + tmpl_v[rr, sl]
            return c2

        lax.fori_loop(0, L, row_body, 0)
        pltpu.sync_copy(obuf_v, out_hbm.at[b])
        return carry

    lax.fori_loop(0, _BPW, per_b, 0)


@functools.partial(jax.jit, static_argnames=())
def kernel(x, W, class_token, pos_encoding):
    xi = x.reshape(B, 2, _HALF).astype(jnp.int32)
    pos = pos_encoding[0, : L + 1]
    ct = class_token.reshape(D)

    mesh = plsc.VectorSubcoreMesh(core_axis_name="c", subcore_axis_name="s")
    f = functools.partial(
        pl.kernel,
        mesh=mesh,
        out_type=jax.ShapeDtypeStruct((B, L + 1, D), jnp.float32),
        scratch_types=[
            pltpu.VMEM((L + 1, D), jnp.float32),
            pltpu.VMEM((D,), jnp.float32),
            pltpu.VMEM((2, _HALF), jnp.int32),
            pltpu.VMEM((L + 1, D), jnp.float32),
            pltpu.SemaphoreType.DMA,
        ],
    )(_sc_body)
    return f(W, xi, pos, ct)


# speedup vs baseline: 4.7497x; 1.4990x over previous
import functools
import math

import jax
import jax.numpy as jnp
from jax import lax
from jax.experimental import pallas as pl
from jax.experimental.pallas import tpu as pltpu, tpu_sc as plsc

B = 1024
L = 200
D = 128
SCALE = math.sqrt(float(D))

_NC = 2
_NS = 16
_NW = _NC * _NS
_BPW = B // _NW
_HALF = L // 2
_NBUF = 3


def _sc_body(w_hbm, xi_hbm, pos_hbm, ct_hbm, out_hbm,
             tmpl_v, ct_v, idx_v, obuf_v, gsem, osem):
    wid = lax.axis_index("s") * _NC + lax.axis_index("c")

    pltpu.sync_copy(xi_hbm.at[wid], idx_v)
    pltpu.sync_copy(pos_hbm, tmpl_v)
    pltpu.sync_copy(ct_hbm, ct_v)
    for c in range(D // 16):
        sl = pl.ds(c * 16, 16)
        tmpl_v[0, sl] = tmpl_v[0, sl] + ct_v[sl]
        for s in range(_NBUF):
            obuf_v[s, 0, sl] = tmpl_v[0, sl]

    def start_gather(i):
        s = i % _NBUF
        return (
            pltpu.async_copy(w_hbm.at[idx_v.at[2 * i]],
                             obuf_v.at[s, pl.ds(1, _HALF)], gsem),
            pltpu.async_copy(w_hbm.at[idx_v.at[2 * i + 1]],
                             obuf_v.at[s, pl.ds(1 + _HALF, _HALF)], gsem),
        )

    for i in range(min(2, _BPW)):
        start_gather(i)

    def wait_write(i):
        s = i % _NBUF
        pltpu.make_async_copy(obuf_v.at[s], out_hbm.at[wid * _BPW + i],
                              osem).wait()

    for i in range(_BPW):
        s = i % _NBUF
        pltpu.make_async_copy(w_hbm.at[idx_v.at[2 * i]],
                              obuf_v.at[s, pl.ds(1, _HALF)], gsem).wait()
        pltpu.make_async_copy(w_hbm.at[idx_v.at[2 * i + 1]],
                              obuf_v.at[s, pl.ds(1 + _HALF, _HALF)],
                              gsem).wait()

        def row_body(r, c2, s=s):
            for dr in range(2):
                rr = 2 * r + 1 + dr
                for c in range(D // 16):
                    sl = pl.ds(c * 16, 16)
                    obuf_v[s, rr, sl] = (obuf_v[s, rr, sl] * SCALE
                                         + tmpl_v[rr, sl])
            return c2

        lax.fori_loop(0, L // 2, row_body, 0)

        pltpu.async_copy(obuf_v.at[s], out_hbm.at[wid * _BPW + i], osem)
        if i + 2 < _BPW:
            if i - 1 >= 0:
                wait_write(i - 1)
            start_gather(i + 2)

    for i in range(_BPW - 3, _BPW):
        wait_write(i)


@functools.partial(jax.jit, static_argnames=())
def kernel(x, W, class_token, pos_encoding):
    xi = x.reshape(_NW, 2 * _BPW, _HALF).astype(jnp.int32)
    pos = pos_encoding[0, : L + 1]
    ct = class_token.reshape(D)

    mesh = plsc.VectorSubcoreMesh(core_axis_name="c", subcore_axis_name="s")
    f = functools.partial(
        pl.kernel,
        mesh=mesh,
        out_type=jax.ShapeDtypeStruct((B, L + 1, D), jnp.float32),
        scratch_types=[
            pltpu.VMEM((L + 1, D), jnp.float32),
            pltpu.VMEM((D,), jnp.float32),
            pltpu.VMEM((2 * _BPW, _HALF), jnp.int32),
            pltpu.VMEM((_NBUF, L + 1, D), jnp.float32),
            pltpu.SemaphoreType.DMA,
            pltpu.SemaphoreType.DMA,
        ],
    )(_sc_body)
    return f(W, xi, pos, ct)


# speedup vs baseline: 8.2157x; 1.7297x over previous
import functools
import math

import jax
import jax.numpy as jnp
from jax import lax
from jax.experimental import pallas as pl
from jax.experimental.pallas import tpu as pltpu, tpu_sc as plsc

B = 1024
L = 200
D = 128
SCALE = math.sqrt(float(D))

_NC = 2
_NS = 16
_NW = _NC * _NS
_Q = 4
_BQ = B // _Q
_NBLK = L * _Q // _NW
_NBUF = 3
_CPW = B // _NW


def _sc_body(w_hbm, xq_hbm, pos_hbm, ct_hbm, out_hbm,
             ct_v, cls_v, posr_v, idx_v, obuf_v, isem, gsem, osem):
    wid = lax.axis_index("s") * _NC + lax.axis_index("c")

    def prefetch(k):
        g = wid * _NBLK + k
        r1 = g // _Q
        q = g % _Q
        a = pltpu.make_async_copy(xq_hbm.at[r1, pl.ds(2 * q, 2)],
                                  idx_v.at[k], isem)
        b = pltpu.make_async_copy(pos_hbm.at[r1 + 1], posr_v.at[k], isem)
        return a, b

    for k in range(_NBLK):
        a, b = prefetch(k)
        a.start()
        b.start()

    pltpu.sync_copy(pos_hbm.at[0], cls_v)
    pltpu.sync_copy(ct_hbm, ct_v)
    for c in range(D // 16):
        sl = pl.ds(c * 16, 16)
        cls_v[sl] = cls_v[sl] + ct_v[sl]
    for b in range(_CPW):
        for c in range(D // 16):
            sl = pl.ds(c * 16, 16)
            obuf_v[0, b, sl] = cls_v[sl]
    pltpu.sync_copy(obuf_v.at[0, pl.ds(0, _CPW)],
                    out_hbm.at[0, pl.ds(wid * _CPW, _CPW)])

    for k in range(_NBLK):
        a, b = prefetch(k)
        a.wait()
        b.wait()

    def start_gather(k):
        s = k % _NBUF
        for j in range(2):
            pltpu.async_copy(w_hbm.at[idx_v.at[k, j]],
                             obuf_v.at[s, pl.ds(j * 128, 128)], gsem)

    def wait_gather(k):
        s = k % _NBUF
        for j in range(2):
            pltpu.make_async_copy(w_hbm.at[idx_v.at[k, j]],
                                  obuf_v.at[s, pl.ds(j * 128, 128)],
                                  gsem).wait()

    def out_dst(k):
        g = wid * _NBLK + k
        return out_hbm.at[g // _Q + 1, pl.ds((g % _Q) * _BQ, _BQ)]

    def wait_write(k):
        pltpu.make_async_copy(obuf_v.at[k % _NBUF], out_dst(k), osem).wait()

    for k in range(min(2, _NBLK)):
        start_gather(k)

    for k in range(_NBLK):
        s = k % _NBUF
        wait_gather(k)

        tvals = [posr_v[k, pl.ds(c * 16, 16)] for c in range(D // 16)]

        def row_body(r, c2, s=s, tvals=tvals):
            for dr in range(2):
                row = 2 * r + dr
                for c in range(D // 16):
                    sl = pl.ds(c * 16, 16)
                    obuf_v[s, row, sl] = (obuf_v[s, row, sl] * SCALE
                                          + tvals[c])
            return c2

        lax.fori_loop(0, _BQ // 2, row_body, 0)

        pltpu.async_copy(obuf_v.at[s], out_dst(k), osem)
        if k + 2 < _NBLK:
            if k >= 1:
                wait_write(k - 1)
            start_gather(k + 2)

    for k in range(_NBLK - 3, _NBLK):
        wait_write(k)


@functools.partial(jax.jit, static_argnames=())
def kernel(x, W, class_token, pos_encoding):
    xq = x.astype(jnp.int32).T.reshape(L, B // 128, 128)
    pos = pos_encoding[0, : L + 1]
    ct = class_token.reshape(D)

    mesh = plsc.VectorSubcoreMesh(core_axis_name="c", subcore_axis_name="s")
    f = functools.partial(
        pl.kernel,
        mesh=mesh,
        out_type=jax.ShapeDtypeStruct((L + 1, B, D), jnp.float32),
        scratch_types=[
            pltpu.VMEM((D,), jnp.float32),
            pltpu.VMEM((D,), jnp.float32),
            pltpu.VMEM((_NBLK, D), jnp.float32),
            pltpu.VMEM((_NBLK, 2, 128), jnp.int32),
            pltpu.VMEM((_NBUF, _BQ, D), jnp.float32),
            pltpu.SemaphoreType.DMA,
            pltpu.SemaphoreType.DMA,
            pltpu.SemaphoreType.DMA,
        ],
    )(_sc_body)
    out = f(W, xq, pos, ct)
    return jnp.transpose(out, (1, 0, 2))
